# baseline (speedup 1.0000x reference)
import jax
import jax.numpy as jnp
from jax.experimental import pallas as pl
from jax.experimental.pallas import tpu as pltpu

B, L, D, DO = 16, 2048, 1024, 128
TLI = 1024
TLT = 256
NSLAB = 3


def _body(n_ref, x_hbm, W1_ref, b1_ref, W2_ref, b2_ref, out_ref, buf, sems):
    def dma(b):
        slot = b % NSLAB
        return pltpu.make_async_copy(
            x_hbm.at[b], buf.at[slot], sems.at[slot])

    for b in range(NSLAB - 1):
        dma(b).start()

    for b in range(B):
        if b + NSLAB - 1 < B:
            dma(b + NSLAB - 1).start()
        dma(b).wait()
        slot = b % NSLAB

        nb = n_ref[b]

        def inner(j, acc, slot=slot):
            xb = buf[slot, pl.ds(j * TLI, TLI), :]
            z = jnp.dot(xb, W1_ref[...], preferred_element_type=jnp.float32)
            y = jnp.maximum(z, -b1_ref[...])
            return acc + jnp.sum(y, axis=0, keepdims=True)

        m = nb // TLI
        acc = jax.lax.fori_loop(
            0, m, inner, jnp.zeros((1, D), jnp.float32))

        def tail(j, acc, slot=slot, nb=nb):
            xb = buf[slot, pl.ds(j * TLT, TLT), :]
            z = jnp.dot(xb, W1_ref[...], preferred_element_type=jnp.float32)
            y = jnp.maximum(z, -b1_ref[...])
            rows = j * TLT + jax.lax.broadcasted_iota(jnp.int32, (TLT, 1), 0)
            y = jnp.where(rows < nb, y, 0.0)
            return acc + jnp.sum(y, axis=0, keepdims=True)

        acc = jax.lax.fori_loop(
            m * (TLI // TLT), (nb + TLT - 1) // TLT, tail, acc)

        pooled = acc / nb.astype(jnp.float32) + b1_ref[...]
        out_ref[pl.ds(b, 1), :] = (
            jnp.dot(pooled, W2_ref[...], preferred_element_type=jnp.float32)
            + b2_ref[...]
        )


def kernel(x, n_instances, W1, b1, W2, b2):
    n = n_instances.astype(jnp.int32)
    b1r = b1.reshape(1, D)
    b2r = b2.reshape(1, DO)

    grid_spec = pltpu.PrefetchScalarGridSpec(
        num_scalar_prefetch=1,
        grid=(1,),
        in_specs=[
            pl.BlockSpec(memory_space=pl.ANY),
            pl.BlockSpec((D, D), lambda *_: (0, 0)),
            pl.BlockSpec((1, D), lambda *_: (0, 0)),
            pl.BlockSpec((D, DO), lambda *_: (0, 0)),
            pl.BlockSpec((1, DO), lambda *_: (0, 0)),
        ],
        out_specs=pl.BlockSpec((B, DO), lambda *_: (0, 0)),
        scratch_shapes=[
            pltpu.VMEM((NSLAB, L, D), jnp.float32),
            pltpu.SemaphoreType.DMA((NSLAB,)),
        ],
    )

    return pl.pallas_call(
        _body,
        grid_spec=grid_spec,
        out_shape=jax.ShapeDtypeStruct((B, DO), jnp.float32),
        compiler_params=pltpu.CompilerParams(
            dimension_semantics=("arbitrary",),
        ),
    )(n, x, W1, b1r, W2, b2r)


# speedup vs baseline: 1.0332x; 1.0332x over previous
import jax
import jax.numpy as jnp
from jax.experimental import pallas as pl
from jax.experimental.pallas import tpu as pltpu

B, L, D, DO = 16, 2048, 1024, 128
TLI = 512
TLT = 256
NSLAB = 4


def _body(n_ref, x_hbm, W1_ref, b1_ref, W2_ref, b2_ref, out_ref, buf, sems):
    def dma(b):
        slot = b % NSLAB
        return pltpu.make_async_copy(
            x_hbm.at[b], buf.at[slot], sems.at[slot])

    for b in range(NSLAB - 1):
        dma(b).start()

    for b in range(B):
        if b + NSLAB - 1 < B:
            dma(b + NSLAB - 1).start()
        dma(b).wait()
        slot = b % NSLAB

        nb = n_ref[b]

        def inner(j, acc, slot=slot):
            xb = buf[slot, pl.ds(j * TLI, TLI), :]
            z = jnp.dot(xb, W1_ref[...], preferred_element_type=jnp.float32)
            y = jnp.maximum(z, -b1_ref[...])
            return acc + jnp.sum(y, axis=0, keepdims=True)

        m = nb // TLI
        acc = jax.lax.fori_loop(
            0, m, inner, jnp.zeros((1, D), jnp.float32))

        def tail(j, acc, slot=slot, nb=nb):
            xb = buf[slot, pl.ds(j * TLT, TLT), :]
            z = jnp.dot(xb, W1_ref[...], preferred_element_type=jnp.float32)
            y = jnp.maximum(z, -b1_ref[...])
            rows = j * TLT + jax.lax.broadcasted_iota(jnp.int32, (TLT, 1), 0)
            y = jnp.where(rows < nb, y, 0.0)
            return acc + jnp.sum(y, axis=0, keepdims=True)

        acc = jax.lax.fori_loop(
            m * (TLI // TLT), (nb + TLT - 1) // TLT, tail, acc)

        pooled = acc / nb.astype(jnp.float32) + b1_ref[...]
        out_ref[pl.ds(b, 1), :] = (
            jnp.dot(pooled, W2_ref[...], preferred_element_type=jnp.float32)
            + b2_ref[...]
        )


def kernel(x, n_instances, W1, b1, W2, b2):
    n = n_instances.astype(jnp.int32)
    b1r = b1.reshape(1, D)
    b2r = b2.reshape(1, DO)

    grid_spec = pltpu.PrefetchScalarGridSpec(
        num_scalar_prefetch=1,
        grid=(1,),
        in_specs=[
            pl.BlockSpec(memory_space=pl.ANY),
            pl.BlockSpec((D, D), lambda *_: (0, 0)),
            pl.BlockSpec((1, D), lambda *_: (0, 0)),
            pl.BlockSpec((D, DO), lambda *_: (0, 0)),
            pl.BlockSpec((1, DO), lambda *_: (0, 0)),
        ],
        out_specs=pl.BlockSpec((B, DO), lambda *_: (0, 0)),
        scratch_shapes=[
            pltpu.VMEM((NSLAB, L, D), jnp.float32),
            pltpu.SemaphoreType.DMA((NSLAB,)),
        ],
    )

    return pl.pallas_call(
        _body,
        grid_spec=grid_spec,
        out_shape=jax.ShapeDtypeStruct((B, DO), jnp.float32),
        compiler_params=pltpu.CompilerParams(
            dimension_semantics=("arbitrary",),
        ),
    )(n, x, W1, b1r, W2, b2r)


# speedup vs baseline: 1.0522x; 1.0184x over previous
import jax
import jax.numpy as jnp
from jax.experimental import pallas as pl
from jax.experimental.pallas import tpu as pltpu

B, L, D, DO = 16, 2048, 1024, 128
TLI = 512
NSLAB = 3


def _body(n_ref, x_hbm, W1_ref, b1_ref, W2_ref, b2_ref, out_ref, buf, sems):
    def dma(b):
        slot = b % NSLAB
        return pltpu.make_async_copy(
            x_hbm.at[b], buf.at[slot], sems.at[slot])

    for b in range(NSLAB - 1):
        dma(b).start()

    for b in range(B):
        if b + NSLAB - 1 < B:
            dma(b + NSLAB - 1).start()
        dma(b).wait()
        slot = b % NSLAB

        nb = n_ref[b]
        jmax = (nb + TLI - 1) // TLI

        def inner(j, acc, slot=slot):
            xb = buf[slot, pl.ds(j * TLI, TLI), :]
            z = jnp.dot(xb, W1_ref[...], preferred_element_type=jnp.float32)
            y = jnp.maximum(z, -b1_ref[...])
            return acc + jnp.sum(y, axis=0, keepdims=True)

        acc = jax.lax.fori_loop(
            0, jmax - 1, inner, jnp.zeros((1, D), jnp.float32))

        xb = buf[slot, pl.ds((jmax - 1) * TLI, TLI), :]
        z = jnp.dot(xb, W1_ref[...], preferred_element_type=jnp.float32)
        y = jnp.maximum(z, -b1_ref[...])
        rows = (jmax - 1) * TLI + jax.lax.broadcasted_iota(
            jnp.int32, (TLI, 1), 0)
        y = jnp.where(rows < nb, y, 0.0)
        acc = acc + jnp.sum(y, axis=0, keepdims=True)

        pooled = acc / nb.astype(jnp.float32) + b1_ref[...]
        out_ref[pl.ds(b, 1), :] = (
            jnp.dot(pooled, W2_ref[...], preferred_element_type=jnp.float32)
            + b2_ref[...]
        )


def kernel(x, n_instances, W1, b1, W2, b2):
    n = n_instances.astype(jnp.int32)
    b1r = b1.reshape(1, D)
    b2r = b2.reshape(1, DO)

    grid_spec = pltpu.PrefetchScalarGridSpec(
        num_scalar_prefetch=1,
        grid=(1,),
        in_specs=[
            pl.BlockSpec(memory_space=pl.ANY),
            pl.BlockSpec((D, D), lambda *_: (0, 0)),
            pl.BlockSpec((1, D), lambda *_: (0, 0)),
            pl.BlockSpec((D, DO), lambda *_: (0, 0)),
            pl.BlockSpec((1, DO), lambda *_: (0, 0)),
        ],
        out_specs=pl.BlockSpec((B, DO), lambda *_: (0, 0)),
        scratch_shapes=[
            pltpu.VMEM((NSLAB, L, D), jnp.float32),
            pltpu.SemaphoreType.DMA((NSLAB,)),
        ],
    )

    return pl.pallas_call(
        _body,
        grid_spec=grid_spec,
        out_shape=jax.ShapeDtypeStruct((B, DO), jnp.float32),
        compiler_params=pltpu.CompilerParams(
            dimension_semantics=("arbitrary",),
        ),
    )(n, x, W1, b1r, W2, b2r)
